# baseline (speedup 1.0000x reference)
import functools

import jax
import jax.numpy as jnp
from jax import lax
from jax.experimental import pallas as pl
from jax.experimental.pallas import tpu as pltpu
from jax.experimental.pallas import tpu_sc as plsc

EMBED_DIM = 128
BATCH = 16384
NUM_CORES = 2
NUM_SUBCORES = 16
NUM_WORKERS = NUM_CORES * NUM_SUBCORES
ROWS_PER_W = BATCH // NUM_WORKERS
CHUNK = 128
NCHUNK = ROWS_PER_W // CHUNK
RING = 3
LANES = 16


def _build_tables_body(res_ref, inc_ref, typ_ref, wrk_ref, w_ref, b_ref,
                       t12_ref, t34_ref):
    w = w_ref[...]
    t1 = jnp.dot(res_ref[...], w[0:8, :], preferred_element_type=jnp.float32)
    t2 = jnp.dot(inc_ref[...], w[8:24, :], preferred_element_type=jnp.float32)
    t3 = jnp.dot(typ_ref[...], w[24:32, :], preferred_element_type=jnp.float32)
    t4 = jnp.dot(wrk_ref[...], w[32:40, :], preferred_element_type=jnp.float32)
    bias = b_ref[...]
    for i in range(20):
        t12_ref[pl.ds(i * 20, 20), :] = t1[i:i + 1, :] + t2
    t4b = t4 + bias
    for i in range(10):
        t34_ref[pl.ds(i * 10, 10), :] = t3[i:i + 1, :] + t4b


def _build_tables(emb_res, emb_inc, emb_typ, emb_wrk, W, b):
    return pl.pallas_call(
        _build_tables_body,
        out_shape=(
            jax.ShapeDtypeStruct((400, EMBED_DIM), jnp.float32),
            jax.ShapeDtypeStruct((100, EMBED_DIM), jnp.float32),
        ),
    )(emb_res, emb_inc, emb_typ, emb_wrk, W, b.reshape(1, EMBED_DIM))


def _sc_lookup(xt, t12, t34):
    mesh = plsc.VectorSubcoreMesh(core_axis_name="c", subcore_axis_name="s")

    @functools.partial(
        pl.kernel,
        mesh=mesh,
        out_type=jax.ShapeDtypeStruct((BATCH, EMBED_DIM), jnp.float32),
        scratch_types=[
            pltpu.VMEM((ROWS_PER_W,), jnp.int32),
            pltpu.VMEM((ROWS_PER_W,), jnp.int32),
            pltpu.VMEM((ROWS_PER_W,), jnp.int32),
            pltpu.VMEM((ROWS_PER_W,), jnp.int32),
            pltpu.VMEM((ROWS_PER_W,), jnp.int32),
            pltpu.VMEM((ROWS_PER_W,), jnp.int32),
        ] + [pltpu.VMEM((CHUNK, EMBED_DIM), jnp.float32)
             for _ in range(RING)]
          + [pltpu.VMEM((CHUNK, EMBED_DIM), jnp.float32)
             for _ in range(RING)]
          + [
            pltpu.VMEM_SHARED((400, EMBED_DIM), jnp.float32),
            pltpu.VMEM_SHARED((100, EMBED_DIM), jnp.float32),
            pltpu.SemaphoreType.DMA,
        ] + [pltpu.SemaphoreType.DMA for _ in range(RING)]
          + [pltpu.SemaphoreType.DMA for _ in range(RING)],
    )
    def k(xth, t12h, t34h, outh,
          x0v, x1v, x2v, x3v, i12v, i34v, *rest):
        b12s = list(rest[0:RING])
        b34s = list(rest[RING:2 * RING])
        t12s, t34s, sx = rest[2 * RING], rest[2 * RING + 1], rest[2 * RING + 2]
        sg = list(rest[2 * RING + 3:2 * RING + 3 + RING])
        so = list(rest[2 * RING + 3 + RING:2 * RING + 3 + 2 * RING])
        sid = lax.axis_index("s")
        wid = sid * NUM_CORES + lax.axis_index("c")
        base = wid * ROWS_PER_W

        xcp = [pltpu.async_copy(xth.at[f, pl.ds(base, ROWS_PER_W)], xv, sx)
               for f, xv in enumerate([x0v, x1v, x2v, x3v])]

        @pl.when(sid == 0)
        def _stage_tables():
            pltpu.sync_copy(t12h, t12s)
            pltpu.sync_copy(t34h, t34s)

        for c in xcp:
            c.wait()
        for r in range(ROWS_PER_W // LANES):
            sl = pl.ds(r * LANES, LANES)
            i12v[sl] = x0v[sl] * 20 + x1v[sl]
            i34v[sl] = x2v[sl] * 10 + x3v[sl]

        plsc.subcore_barrier()

        def issue(c):
            s = c % RING
            isl = pl.ds(c * CHUNK, CHUNK)
            return (pltpu.async_copy(t12s.at[i12v.at[isl]], b12s[s], sg[s]),
                    pltpu.async_copy(t34s.at[i34v.at[isl]], b34s[s], sg[s]))

        UNROLL = 8
        LOOKAHEAD = RING - 1
        gcp = [None] * NCHUNK
        ocp = [None] * NCHUNK
        for c0 in range(min(LOOKAHEAD, NCHUNK)):
            gcp[c0] = issue(c0)
        for c in range(NCHUNK):
            s = c % RING
            n = c + LOOKAHEAD
            if n < NCHUNK:
                if n - RING >= 0:
                    ocp[n - RING].wait()
                gcp[n] = issue(n)
            gcp[c][0].wait()
            gcp[c][1].wait()
            b12, b34 = b12s[s], b34s[s]

            def add_body(r, carry):
                for u in range(UNROLL):
                    for jj in range(EMBED_DIM // LANES):
                        sl = pl.ds(jj * LANES, LANES)
                        plsc.addupdate(b12.at[r * UNROLL + u, sl],
                                       b34[r * UNROLL + u, sl])
                return carry

            lax.fori_loop(0, CHUNK // UNROLL, add_body, 0)
            ocp[c] = pltpu.async_copy(
                b12, outh.at[pl.ds(base + c * CHUNK, CHUNK)], so[s])
        for c in range(max(0, NCHUNK - RING), NCHUNK):
            ocp[c].wait()

    return k(xt, t12, t34)


def kernel(x, emb_res, emb_inc, emb_typ, emb_wrk, W, b):
    t12, t34 = _build_tables(emb_res, emb_inc, emb_typ, emb_wrk, W, b)
    out = _sc_lookup(x.astype(jnp.int32).T, t12, t34)
    return out[:, None, :]


# speedup vs baseline: 1.0132x; 1.0132x over previous
import functools

import jax
import jax.numpy as jnp
from jax import lax
from jax.experimental import pallas as pl
from jax.experimental.pallas import tpu as pltpu
from jax.experimental.pallas import tpu_sc as plsc

EMBED_DIM = 128
BATCH = 16384
NUM_CORES = 2
NUM_SUBCORES = 16
NUM_WORKERS = NUM_CORES * NUM_SUBCORES
ROWS_PER_W = BATCH // NUM_WORKERS
CHUNK = 128
NCHUNK = ROWS_PER_W // CHUNK
RING = 3
LANES = 16


def _build_tables_body(res_ref, inc_ref, typ_ref, wrk_ref, w_ref, b_ref,
                       t12_ref, t34_ref):
    w = w_ref[...]
    t1 = jnp.dot(res_ref[...], w[0:8, :], preferred_element_type=jnp.float32)
    t2 = jnp.dot(inc_ref[...], w[8:24, :], preferred_element_type=jnp.float32)
    t3 = jnp.dot(typ_ref[...], w[24:32, :], preferred_element_type=jnp.float32)
    t4 = jnp.dot(wrk_ref[...], w[32:40, :], preferred_element_type=jnp.float32)
    bias = b_ref[...]
    for i in range(20):
        t12_ref[pl.ds(i * 20, 20), :] = t1[i:i + 1, :] + t2
    t4b = t4 + bias
    for i in range(10):
        t34_ref[pl.ds(i * 10, 10), :] = t3[i:i + 1, :] + t4b


def _build_tables(emb_res, emb_inc, emb_typ, emb_wrk, W, b):
    return pl.pallas_call(
        _build_tables_body,
        out_shape=(
            jax.ShapeDtypeStruct((400, EMBED_DIM), jnp.float32),
            jax.ShapeDtypeStruct((100, EMBED_DIM), jnp.float32),
        ),
    )(emb_res, emb_inc, emb_typ, emb_wrk, W, b.reshape(1, EMBED_DIM))


def _sc_lookup(xt, t12, t34):
    mesh = plsc.VectorSubcoreMesh(core_axis_name="c", subcore_axis_name="s")

    @functools.partial(
        pl.kernel,
        mesh=mesh,
        out_type=jax.ShapeDtypeStruct((BATCH, EMBED_DIM), jnp.float32),
        scratch_types=[
            pltpu.VMEM((ROWS_PER_W,), jnp.int32),
            pltpu.VMEM((ROWS_PER_W,), jnp.int32),
            pltpu.VMEM((ROWS_PER_W,), jnp.int32),
            pltpu.VMEM((ROWS_PER_W,), jnp.int32),
            pltpu.VMEM((ROWS_PER_W,), jnp.int32),
            pltpu.VMEM((ROWS_PER_W,), jnp.int32),
        ] + [pltpu.VMEM((CHUNK, EMBED_DIM), jnp.float32)
             for _ in range(RING)]
          + [pltpu.VMEM((CHUNK, EMBED_DIM), jnp.float32)
             for _ in range(RING)]
          + [
            pltpu.VMEM_SHARED((400, EMBED_DIM), jnp.float32),
            pltpu.VMEM_SHARED((100, EMBED_DIM), jnp.float32),
            pltpu.SemaphoreType.DMA,
        ] + [pltpu.SemaphoreType.DMA for _ in range(RING)]
          + [pltpu.SemaphoreType.DMA for _ in range(RING)],
    )
    def k(xth, t12h, t34h, outh,
          x0v, x1v, x2v, x3v, i12v, i34v, *rest):
        b12s = list(rest[0:RING])
        b34s = list(rest[RING:2 * RING])
        t12s, t34s, sx = rest[2 * RING], rest[2 * RING + 1], rest[2 * RING + 2]
        sg = list(rest[2 * RING + 3:2 * RING + 3 + RING])
        so = list(rest[2 * RING + 3 + RING:2 * RING + 3 + 2 * RING])
        sid = lax.axis_index("s")
        wid = sid * NUM_CORES + lax.axis_index("c")
        base = wid * ROWS_PER_W

        xcp = [pltpu.async_copy(xth.at[f, pl.ds(base, ROWS_PER_W)], xv, sx)
               for f, xv in enumerate([x0v, x1v, x2v, x3v])]

        @pl.when(sid == 0)
        def _stage_tables():
            pltpu.sync_copy(t12h, t12s)
            pltpu.sync_copy(t34h, t34s)

        for c in xcp:
            c.wait()
        for r in range(ROWS_PER_W // LANES):
            sl = pl.ds(r * LANES, LANES)
            i12v[sl] = x0v[sl] * 20 + x1v[sl]
            i34v[sl] = x2v[sl] * 10 + x3v[sl]

        plsc.subcore_barrier()

        def issue(c):
            s = c % RING
            isl = pl.ds(c * CHUNK, CHUNK)
            return (pltpu.async_copy(t12s.at[i12v.at[isl]], b12s[s], sg[s]),
                    pltpu.async_copy(t34s.at[i34v.at[isl]], b34s[s], sg[s]))

        UNROLL = 4
        LOOKAHEAD = RING - 1
        gcp = [None] * NCHUNK
        ocp = [None] * NCHUNK
        for c0 in range(min(LOOKAHEAD, NCHUNK)):
            gcp[c0] = issue(c0)
        for c in range(NCHUNK):
            s = c % RING
            n = c + LOOKAHEAD
            if n < NCHUNK:
                if n - RING >= 0:
                    ocp[n - RING].wait()
                gcp[n] = issue(n)
            gcp[c][0].wait()
            gcp[c][1].wait()
            b12, b34 = b12s[s], b34s[s]

            def add_body(r, carry):
                for u in range(UNROLL):
                    for jj in range(EMBED_DIM // LANES):
                        sl = pl.ds(jj * LANES, LANES)
                        plsc.addupdate(b12.at[r * UNROLL + u, sl],
                                       b34[r * UNROLL + u, sl])
                return carry

            lax.fori_loop(0, CHUNK // UNROLL, add_body, 0)
            ocp[c] = pltpu.async_copy(
                b12, outh.at[pl.ds(base + c * CHUNK, CHUNK)], so[s])
        for c in range(max(0, NCHUNK - RING), NCHUNK):
            ocp[c].wait()

    return k(xt, t12, t34)


def kernel(x, emb_res, emb_inc, emb_typ, emb_wrk, W, b):
    t12, t34 = _build_tables(emb_res, emb_inc, emb_typ, emb_wrk, W, b)
    out = _sc_lookup(x.astype(jnp.int32).T, t12, t34)
    return out[:, None, :]
